# baseline (speedup 1.0000x reference)
import functools

import jax
import jax.numpy as jnp
from jax.experimental import pallas as pl
from jax.experimental.pallas import tpu as pltpu

BEAM = 8
VOCAB = 1_000_000
CHUNK = 65_536
NCHUNK = 16
KCAND = 10

_NEG = float("-inf")
_BIGI = 2**30


def _extract_topk(vals, idxs, k):
    vs, is_ = [], []
    for _ in range(k):
        m = jnp.max(vals, axis=1, keepdims=True)
        hit = vals == m
        sel = jnp.min(jnp.where(hit, idxs, _BIGI), axis=1, keepdims=True)
        vs.append(m)
        is_.append(sel)
        kill = (idxs == sel) & hit
        vals = jnp.where(kill, _NEG, vals)
    return vs, is_


def _body(d_ref, gen_ref, scores_ref, step_ref, gen_out_ref, sc_out_ref,
          cand_v_ref, cand_i_ref, minacc_ref):
    i = pl.program_id(0)

    d = d_ref[...]
    gcol = jax.lax.broadcasted_iota(jnp.int32, (BEAM, CHUNK), 1) + i * CHUNK
    valid = gcol < VOCAB
    dmax = jnp.where(valid, d, _NEG)
    dmin = jnp.where(valid, d, float("inf"))

    minc = jnp.min(dmin, axis=1, keepdims=True)

    @pl.when(i == 0)
    def _():
        minacc_ref[...] = jnp.full((BEAM, 1), jnp.inf, jnp.float32)

    minacc_ref[...] = jnp.minimum(minacc_ref[...], minc)

    vs, is_ = _extract_topk(dmax, gcol, KCAND)
    pad = 128 - KCAND
    vrow = jnp.concatenate(vs + [jnp.full((BEAM, pad), _NEG)], axis=1)
    irow = jnp.concatenate(is_ + [jnp.full((BEAM, pad), _BIGI, jnp.int32)], axis=1)
    cand_v_ref[i] = vrow
    cand_i_ref[i] = irow

    @pl.when(i == NCHUNK - 1)
    def _():
        gen = gen_ref[...]
        step = step_ref[0, 0]
        col8 = jax.lax.broadcasted_iota(jnp.int32, (BEAM, 128), 1)
        p0 = jnp.sum(jnp.where(col8 == step - 2, gen, 0), axis=1, keepdims=True)
        p1 = jnp.sum(jnp.where(col8 == step - 1, gen, 0), axis=1, keepdims=True)

        cv = jnp.concatenate([cand_v_ref[c] for c in range(NCHUNK)], axis=1)
        ci = jnp.concatenate([cand_i_ref[c] for c in range(NCHUNK)], axis=1)
        disq = (ci == p0) | (ci == p1)
        cv = jnp.where(disq, _NEG, cv)
        ci = jnp.where(disq, _BIGI, ci)

        minv = minacc_ref[...]
        cv = jnp.concatenate([cv, minv, minv], axis=1)
        ci = jnp.concatenate([ci, p0, p1], axis=1)

        v8, i8 = _extract_topk(cv, ci, BEAM)
        vals8 = jnp.concatenate(v8, axis=1)
        idx8 = jnp.concatenate(i8, axis=1)

        sc = jnp.log(vals8) + scores_ref[...]
        fr = jax.lax.broadcasted_iota(jnp.int32, (BEAM, BEAM), 0)
        fc = jax.lax.broadcasted_iota(jnp.int32, (BEAM, BEAM), 1)
        fi = fr * BEAM + fc

        row8 = jax.lax.broadcasted_iota(jnp.int32, (BEAM, 128), 0)
        for j in range(BEAM):
            m = jnp.max(sc)
            sel = jnp.min(jnp.where(sc == m, fi, 64))
            r = sel // BEAM
            tok = jnp.sum(jnp.where(fi == sel, idx8, 0))
            sc_out_ref[j:j + 1, 0:1] = jnp.broadcast_to(m, (1, 1))
            sc = jnp.where(fi == sel, _NEG, sc)

            rowsel = jnp.sum(jnp.where(row8 == r, gen, 0), axis=0, keepdims=True)
            base = gen[j:j + 1, :]
            colv = col8[0:1, :]
            newrow = jnp.where(colv < step, rowsel, base)
            newrow = jnp.where(colv == step, tok, newrow)
            gen_out_ref[j:j + 1, :] = newrow


@jax.jit
def _run(gen_seq, d, scores, step):
    gen_new, sc_new = pl.pallas_call(
        _body,
        grid=(NCHUNK,),
        in_specs=[
            pl.BlockSpec((BEAM, CHUNK), lambda i: (0, i)),
            pl.BlockSpec((BEAM, 128), lambda i: (0, 0)),
            pl.BlockSpec((BEAM, 1), lambda i: (0, 0)),
            pl.BlockSpec(memory_space=pltpu.SMEM),
        ],
        out_specs=[
            pl.BlockSpec((BEAM, 128), lambda i: (0, 0)),
            pl.BlockSpec((BEAM, 1), lambda i: (0, 0)),
        ],
        out_shape=[
            jax.ShapeDtypeStruct((BEAM, 128), jnp.int32),
            jax.ShapeDtypeStruct((BEAM, 1), jnp.float32),
        ],
        scratch_shapes=[
            pltpu.VMEM((NCHUNK, BEAM, 128), jnp.float32),
            pltpu.VMEM((NCHUNK, BEAM, 128), jnp.int32),
            pltpu.VMEM((BEAM, 1), jnp.float32),
        ],
    )(d, gen_seq, scores, step)
    return gen_new, sc_new


def kernel(gen_seq, dec_output, scores, step):
    d = dec_output[:, -1, :]
    scores2 = scores.reshape(BEAM, 1)
    step_arr = jnp.asarray(step, jnp.int32).reshape(1, 1)
    gen_new, sc_new = _run(gen_seq, d, scores2, step_arr)
    return gen_new, sc_new.reshape(BEAM)
